# baseline (speedup 1.0000x reference)
import functools

import jax
import jax.numpy as jnp
from jax import lax
from jax.experimental import pallas as pl
from jax.experimental.pallas import tpu as pltpu
from jax.experimental.pallas import tpu_sc as plsc

N = 10000
E = 320000
D = 128

NC = 2
NS = 16
NW = NC * NS
EPW = E // NW
BK = 80
NBLK = EPW // BK
BKA = 40
NBLKA = EPW // BKA
SR = 624
TAIL = N - NS * SR
ZR = 48

_MESH = dict(core_axis_name="c", subcore_axis_name="s")


def _linear(x, W, b):
    def body(x_ref, w_ref, b_ref, o_ref):
        o_ref[...] = (
            jnp.dot(x_ref[...], w_ref[...],
                    preferred_element_type=jnp.float32,
                    precision=lax.Precision.HIGHEST)
            + b_ref[...]
        )

    return pl.pallas_call(
        body,
        grid=(10,),
        in_specs=[
            pl.BlockSpec((N // 10, D), lambda i: (i, 0)),
            pl.BlockSpec((D, D), lambda i: (0, 0)),
            pl.BlockSpec((1, D), lambda i: (0, 0)),
        ],
        out_specs=pl.BlockSpec((N // 10, D), lambda i: (i, 0)),
        out_shape=jax.ShapeDtypeStruct((N, D), jnp.float32),
    )(x, W, b.reshape(1, D))


def _combine_relu_linear(p, W, b):
    def body(p_ref, w_ref, b_ref, o_ref):
        h = jnp.maximum(p_ref[0] + p_ref[1], 0.0)
        o_ref[...] = (
            jnp.dot(h, w_ref[...],
                    preferred_element_type=jnp.float32,
                    precision=lax.Precision.HIGHEST)
            + b_ref[...]
        )

    return pl.pallas_call(
        body,
        grid=(10,),
        in_specs=[
            pl.BlockSpec((2, N // 10, D), lambda i: (0, i, 0)),
            pl.BlockSpec((D, D), lambda i: (0, 0)),
            pl.BlockSpec((1, D), lambda i: (0, 0)),
        ],
        out_specs=pl.BlockSpec((N // 10, D), lambda i: (i, 0)),
        out_shape=jax.ShapeDtypeStruct((N, D), jnp.float32),
    )(p, W, b.reshape(1, D))


def _combine(p):
    def body(p_ref, o_ref):
        o_ref[...] = p_ref[0] + p_ref[1]

    return pl.pallas_call(
        body,
        grid=(10,),
        in_specs=[pl.BlockSpec((2, N // 10, D), lambda i: (0, i, 0))],
        out_specs=pl.BlockSpec((N // 10, D), lambda i: (i, 0)),
        out_shape=jax.ShapeDtypeStruct((N, D), jnp.float32),
    )(p)


def _sc_aggregate(support, row, col, vals):
    mesh = plsc.VectorSubcoreMesh(**_MESH)

    @functools.partial(
        pl.kernel,
        mesh=mesh,
        compiler_params=pltpu.CompilerParams(needs_layout_passes=False),
        out_type=jax.ShapeDtypeStruct((NC, N, D), jnp.float32),
        scratch_types=[
            pltpu.VMEM((EPW,), jnp.int32),
            pltpu.VMEM((EPW,), jnp.float32),
            pltpu.VMEM((EPW,), jnp.int32),
            pltpu.VMEM((2, BKA), jnp.int32),
            pltpu.VMEM((BKA, D), jnp.float32),
            pltpu.VMEM((BKA, D), jnp.float32),
            pltpu.VMEM((BKA, D), jnp.float32),
            pltpu.VMEM((BKA, D), jnp.float32),
            pltpu.VMEM_SHARED((N, D), jnp.float32),
            pltpu.SemaphoreType.DMA,
            pltpu.SemaphoreType.DMA,
            pltpu.SemaphoreType.DMA,
            pltpu.SemaphoreType.DMA,
            pltpu.SemaphoreType.DMA,
        ],
    )
    def agg(support_hbm, row_hbm, col_hbm, vals_hbm, out_hbm,
            colv, valsv, rowp, rowv, gA, gB, sA, sB, hsh,
            semA, semB, semSA, semSB, semI):
        c = lax.axis_index("c")
        s = lax.axis_index("s")
        wid = c * NS + s
        e0 = wid * EPW

        pltpu.async_copy(col_hbm.at[pl.ds(e0, EPW)], colv, semI)
        pltpu.async_copy(vals_hbm.at[pl.ds(e0, EPW)], valsv, semI)
        pltpu.async_copy(row_hbm.at[pl.ds(e0, EPW)], rowp, semI)

        def zb_body(i, carry):
            sA[i // 8, pl.ds((i % 8) * 16, 16)] = jnp.zeros((16,), jnp.float32)
            return carry
        lax.fori_loop(0, BKA * 8, zb_body, 0)
        base_r = s * SR
        for j in range(SR // BKA):
            pltpu.sync_copy(sA, hsh.at[pl.ds(base_r + j * BKA, BKA)])
        pltpu.sync_copy(sA.at[pl.ds(0, SR % BKA)],
                        hsh.at[pl.ds(base_r + SR - SR % BKA, SR % BKA)])

        @pl.when(s == NS - 1)
        def _zero_tail():
            pltpu.sync_copy(sA.at[pl.ds(0, TAIL)], hsh.at[pl.ds(NS * SR, TAIL)])

        pltpu.make_async_copy(col_hbm.at[pl.ds(e0, EPW)], colv, semI).wait()
        pltpu.make_async_copy(vals_hbm.at[pl.ds(e0, EPW)], valsv, semI).wait()
        pltpu.make_async_copy(row_hbm.at[pl.ds(e0, EPW)], rowp, semI).wait()
        plsc.subcore_barrier()

        def issue_gather(g, sem, blk_i):
            pltpu.async_copy(
                support_hbm.at[colv.at[pl.ds(blk_i * BKA, BKA)]], g, sem)

        def wait_gather(g, sem, blk_i):
            pltpu.make_async_copy(
                support_hbm.at[colv.at[pl.ds(blk_i * BKA, BKA)]], g, sem).wait()

        def scale(g, sbuf, blk_i):
            for q0, t0 in ((0, 0), (16, 0), (BKA - 16, 8)):
                vch = valsv[pl.ds(blk_i * BKA + q0, 16)]
                for t in range(t0, 16):
                    j = q0 + t
                    vsp = jnp.full((16,), vch[t])
                    for k in range(D // 16):
                        sl = pl.ds(k * 16, 16)
                        sbuf[j, sl] = g[j, sl] * vsp

        def copy_rows(slot, blk_i):
            rowv[slot, pl.ds(0, 16)] = rowp[pl.ds(blk_i * BKA, 16)]
            rowv[slot, pl.ds(16, 16)] = rowp[pl.ds(blk_i * BKA + 16, 16)]
            rowv[slot, pl.ds(BKA - 16, 16)] = rowp[pl.ds(blk_i * BKA + BKA - 16, 16)]

        def issue_scatter(sbuf, slot, sem, blk_i):
            copy_rows(slot, blk_i)
            pltpu.async_copy(sbuf, hsh.at[rowv.at[slot]], sem, add=True)

        def wait_scatter(sbuf, slot, sem):
            pltpu.make_async_copy(sbuf, hsh.at[rowv.at[slot]], sem).wait()

        issue_gather(gA, semA, 0)
        issue_gather(gB, semB, 1)

        def body2(i, carry):
            bA = 2 * i
            wait_gather(gA, semA, bA)

            @pl.when(i >= 1)
            def _drain_a():
                wait_scatter(sA, 0, semSA)
            scale(gA, sA, bA)

            @pl.when(i < NBLKA // 2 - 1)
            def _next_a():
                issue_gather(gA, semA, bA + 2)
            issue_scatter(sA, 0, semSA, bA)

            wait_gather(gB, semB, bA + 1)

            @pl.when(i >= 1)
            def _drain_b():
                wait_scatter(sB, 1, semSB)
            scale(gB, sB, bA + 1)

            @pl.when(i < NBLKA // 2 - 1)
            def _next_b():
                issue_gather(gB, semB, bA + 3)
            issue_scatter(sB, 1, semSB, bA + 1)
            return carry
        lax.fori_loop(0, NBLKA // 2, body2, 0)
        wait_scatter(sA, 0, semSA)
        wait_scatter(sB, 1, semSB)
        plsc.subcore_barrier()

        pltpu.sync_copy(hsh.at[pl.ds(base_r, SR)],
                        out_hbm.at[c, pl.ds(base_r, SR)])

        @pl.when(s == NS - 1)
        def _write_tail():
            pltpu.sync_copy(hsh.at[pl.ds(NS * SR, TAIL)],
                            out_hbm.at[c, pl.ds(NS * SR, TAIL)])

    return agg(support, row, col, vals)


def _sc_decode(z, src, dst):
    mesh = plsc.VectorSubcoreMesh(**_MESH)

    @functools.partial(
        pl.kernel,
        mesh=mesh,
        compiler_params=pltpu.CompilerParams(needs_layout_passes=False),
        out_type=jax.ShapeDtypeStruct((E,), jnp.float32),
        scratch_types=[
            pltpu.VMEM((EPW,), jnp.int32),
            pltpu.VMEM((EPW,), jnp.int32),
            pltpu.VMEM((BK, D), jnp.float32),
            pltpu.VMEM((BK, D), jnp.float32),
            pltpu.VMEM((BK, D), jnp.float32),
            pltpu.VMEM((BK, D), jnp.float32),
            pltpu.VMEM((16, 16), jnp.float32),
            pltpu.VMEM((BK,), jnp.float32),
            pltpu.SemaphoreType.DMA,
            pltpu.SemaphoreType.DMA,
            pltpu.SemaphoreType.DMA,
        ],
    )
    def dec(z_hbm, src_hbm, dst_hbm, out_hbm,
            sv, dv, zaA, zbA, zaB, zbB, pbuf, res, semA, semB, semI):
        c = lax.axis_index("c")
        s = lax.axis_index("s")
        wid = c * NS + s
        e0 = wid * EPW
        lanes = lax.iota(jnp.int32, 16)

        pltpu.async_copy(src_hbm.at[pl.ds(e0, EPW)], sv, semI)
        pltpu.async_copy(dst_hbm.at[pl.ds(e0, EPW)], dv, semI)
        pltpu.make_async_copy(src_hbm.at[pl.ds(e0, EPW)], sv, semI).wait()
        pltpu.make_async_copy(dst_hbm.at[pl.ds(e0, EPW)], dv, semI).wait()

        def issue(za, zb, sem, blk_i):
            pltpu.async_copy(z_hbm.at[sv.at[pl.ds(blk_i * BK, BK)]], za, sem)
            pltpu.async_copy(z_hbm.at[dv.at[pl.ds(blk_i * BK, BK)]], zb, sem)

        def drain(za, zb, sem, blk_i):
            pltpu.make_async_copy(
                z_hbm.at[sv.at[pl.ds(blk_i * BK, BK)]], za, sem).wait()
            pltpu.make_async_copy(
                z_hbm.at[dv.at[pl.ds(blk_i * BK, BK)]], zb, sem).wait()

        def _tree_sum(terms):
            while len(terms) > 1:
                terms = [terms[a] + terms[a + 1]
                         for a in range(0, len(terms) - 1, 2)] + (
                    [terms[-1]] if len(terms) % 2 else [])
            return terms[0]

        def compute(za, zb, blk_i):
            def grp(g5, carry):
                for t in range(16):
                    j = g5 * 16 + t
                    pbuf[t, pl.ds(0, 16)] = _tree_sum(
                        [za[j, pl.ds(k * 16, 16)] * zb[j, pl.ds(k * 16, 16)]
                         for k in range(D // 16)])
                tot = _tree_sum(
                    [plsc.load_gather(pbuf, [lanes, jnp.full((16,), d2, jnp.int32)])
                     for d2 in range(16)])
                res[pl.ds(g5 * 16, 16)] = tot
                return carry
            lax.fori_loop(0, BK // 16, grp, 0)
            pltpu.sync_copy(res, out_hbm.at[pl.ds(e0 + blk_i * BK, BK)])

        issue(zaA, zbA, semA, 0)
        issue(zaB, zbB, semB, 1)

        def body2(i, carry):
            bA = 2 * i
            drain(zaA, zbA, semA, bA)
            compute(zaA, zbA, bA)
            issue(zaA, zbA, semA, bA + 2)
            drain(zaB, zbB, semB, bA + 1)
            compute(zaB, zbB, bA + 1)

            @pl.when(i < (NBLK - 3) // 2)
            def _next_b():
                issue(zaB, zbB, semB, bA + 3)
            return carry
        lax.fori_loop(0, (NBLK - 1) // 2, body2, 0)
        drain(zaA, zbA, semA, NBLK - 1)
        compute(zaA, zbA, NBLK - 1)

    return dec(z, src, dst)


def kernel(x, adj_sparse_indices, adj_sparse_values, edge_index, W1, b1, W2, b2):
    row = adj_sparse_indices[0]
    col = adj_sparse_indices[1]
    src = edge_index[0]
    dst = edge_index[1]

    support = _linear(x, W1, b1)
    hp = _sc_aggregate(support, row, col, adj_sparse_values)
    support2 = _combine_relu_linear(hp, W2, b2)
    zp = _sc_aggregate(support2, row, col, adj_sparse_values)
    z = _combine(zp)
    return _sc_decode(z, src, dst)


# speedup vs baseline: 1.0409x; 1.0409x over previous
import functools

import jax
import jax.numpy as jnp
from jax import lax
from jax.experimental import pallas as pl
from jax.experimental.pallas import tpu as pltpu
from jax.experimental.pallas import tpu_sc as plsc

N = 10000
E = 320000
D = 128

NC = 2
NS = 16
NW = NC * NS
EPW = E // NW
BK = 80
NBLK = EPW // BK
BKA = 40
NBLKA = EPW // BKA
SR = 624
TAIL = N - NS * SR
ZR = 48

_MESH = dict(core_axis_name="c", subcore_axis_name="s")


def _linear(x, W, b):
    def body(x_ref, w_ref, b_ref, o_ref):
        o_ref[...] = (
            jnp.dot(x_ref[...], w_ref[...],
                    preferred_element_type=jnp.float32,
                    precision=lax.Precision.HIGHEST)
            + b_ref[...]
        )

    return pl.pallas_call(
        body,
        grid=(10,),
        in_specs=[
            pl.BlockSpec((N // 10, D), lambda i: (i, 0)),
            pl.BlockSpec((D, D), lambda i: (0, 0)),
            pl.BlockSpec((1, D), lambda i: (0, 0)),
        ],
        out_specs=pl.BlockSpec((N // 10, D), lambda i: (i, 0)),
        out_shape=jax.ShapeDtypeStruct((N, D), jnp.float32),
    )(x, W, b.reshape(1, D))


def _combine_relu_linear(p, W, b):
    def body(p_ref, w_ref, b_ref, o_ref):
        h = jnp.maximum(p_ref[0] + p_ref[1], 0.0)
        o_ref[...] = (
            jnp.dot(h, w_ref[...],
                    preferred_element_type=jnp.float32,
                    precision=lax.Precision.HIGHEST)
            + b_ref[...]
        )

    return pl.pallas_call(
        body,
        grid=(10,),
        in_specs=[
            pl.BlockSpec((2, N // 10, D), lambda i: (0, i, 0)),
            pl.BlockSpec((D, D), lambda i: (0, 0)),
            pl.BlockSpec((1, D), lambda i: (0, 0)),
        ],
        out_specs=pl.BlockSpec((N // 10, D), lambda i: (i, 0)),
        out_shape=jax.ShapeDtypeStruct((N, D), jnp.float32),
    )(p, W, b.reshape(1, D))


def _combine(p):
    def body(p_ref, o_ref):
        o_ref[...] = p_ref[0] + p_ref[1]

    return pl.pallas_call(
        body,
        grid=(10,),
        in_specs=[pl.BlockSpec((2, N // 10, D), lambda i: (0, i, 0))],
        out_specs=pl.BlockSpec((N // 10, D), lambda i: (i, 0)),
        out_shape=jax.ShapeDtypeStruct((N, D), jnp.float32),
    )(p)


def _sc_aggregate(support, row, col, vals):
    mesh = plsc.VectorSubcoreMesh(**_MESH)

    @functools.partial(
        pl.kernel,
        mesh=mesh,
        compiler_params=pltpu.CompilerParams(needs_layout_passes=False),
        out_type=jax.ShapeDtypeStruct((NC, N, D), jnp.float32),
        scratch_types=[
            pltpu.VMEM((EPW,), jnp.int32),
            pltpu.VMEM((EPW,), jnp.float32),
            pltpu.VMEM((EPW,), jnp.int32),
            pltpu.VMEM((2, BKA), jnp.int32),
            pltpu.VMEM((BKA, D), jnp.float32),
            pltpu.VMEM((BKA, D), jnp.float32),
            pltpu.VMEM((BKA, D), jnp.float32),
            pltpu.VMEM((BKA, D), jnp.float32),
            pltpu.VMEM_SHARED((N, D), jnp.float32),
            pltpu.SemaphoreType.DMA,
            pltpu.SemaphoreType.DMA,
            pltpu.SemaphoreType.DMA,
            pltpu.SemaphoreType.DMA,
            pltpu.SemaphoreType.DMA,
        ],
    )
    def agg(support_hbm, row_hbm, col_hbm, vals_hbm, out_hbm,
            colv, valsv, rowp, rowv, gA, gB, sA, sB, hsh,
            semA, semB, semSA, semSB, semI):
        c = lax.axis_index("c")
        s = lax.axis_index("s")
        wid = c * NS + s
        e0 = wid * EPW

        pltpu.async_copy(col_hbm.at[pl.ds(e0, EPW)], colv, semI)
        pltpu.async_copy(vals_hbm.at[pl.ds(e0, EPW)], valsv, semI)
        pltpu.async_copy(row_hbm.at[pl.ds(e0, EPW)], rowp, semI)

        def zb_body(i, carry):
            sA[i // 8, pl.ds((i % 8) * 16, 16)] = jnp.zeros((16,), jnp.float32)
            return carry
        lax.fori_loop(0, BKA * 8, zb_body, 0)
        base_r = s * SR
        for j in range(SR // BKA):
            pltpu.sync_copy(sA, hsh.at[pl.ds(base_r + j * BKA, BKA)])
        pltpu.sync_copy(sA.at[pl.ds(0, SR % BKA)],
                        hsh.at[pl.ds(base_r + SR - SR % BKA, SR % BKA)])

        @pl.when(s == NS - 1)
        def _zero_tail():
            pltpu.sync_copy(sA.at[pl.ds(0, TAIL)], hsh.at[pl.ds(NS * SR, TAIL)])

        pltpu.make_async_copy(col_hbm.at[pl.ds(e0, EPW)], colv, semI).wait()
        pltpu.make_async_copy(vals_hbm.at[pl.ds(e0, EPW)], valsv, semI).wait()
        pltpu.make_async_copy(row_hbm.at[pl.ds(e0, EPW)], rowp, semI).wait()
        plsc.subcore_barrier()

        def issue_gather(g, sem, blk_i):
            pltpu.async_copy(
                support_hbm.at[colv.at[pl.ds(blk_i * BKA, BKA)]], g, sem)

        def wait_gather(g, sem, blk_i):
            pltpu.make_async_copy(
                support_hbm.at[colv.at[pl.ds(blk_i * BKA, BKA)]], g, sem).wait()

        def scale(g, sbuf, blk_i):
            for q0, t0 in ((0, 0), (16, 0), (BKA - 16, 8)):
                vch = valsv[pl.ds(blk_i * BKA + q0, 16)]
                for t in range(t0, 16):
                    j = q0 + t
                    vsp = jnp.full((16,), vch[t])
                    for k in range(D // 16):
                        sl = pl.ds(k * 16, 16)
                        sbuf[j, sl] = g[j, sl] * vsp

        def copy_rows(slot, blk_i):
            rowv[slot, pl.ds(0, 16)] = rowp[pl.ds(blk_i * BKA, 16)]
            rowv[slot, pl.ds(16, 16)] = rowp[pl.ds(blk_i * BKA + 16, 16)]
            rowv[slot, pl.ds(BKA - 16, 16)] = rowp[pl.ds(blk_i * BKA + BKA - 16, 16)]

        def issue_scatter(sbuf, slot, sem, blk_i):
            copy_rows(slot, blk_i)
            pltpu.async_copy(sbuf, hsh.at[rowv.at[slot]], sem, add=True)

        def wait_scatter(sbuf, slot, sem):
            pltpu.make_async_copy(sbuf, hsh.at[rowv.at[slot]], sem).wait()

        issue_gather(gA, semA, 0)
        issue_gather(gB, semB, 1)

        def body2(i, carry):
            bA = 2 * i
            wait_gather(gA, semA, bA)

            @pl.when(i >= 1)
            def _drain_a():
                wait_scatter(sA, 0, semSA)
            scale(gA, sA, bA)

            @pl.when(i < NBLKA // 2 - 1)
            def _next_a():
                issue_gather(gA, semA, bA + 2)
            issue_scatter(sA, 0, semSA, bA)

            wait_gather(gB, semB, bA + 1)

            @pl.when(i >= 1)
            def _drain_b():
                wait_scatter(sB, 1, semSB)
            scale(gB, sB, bA + 1)

            @pl.when(i < NBLKA // 2 - 1)
            def _next_b():
                issue_gather(gB, semB, bA + 3)
            issue_scatter(sB, 1, semSB, bA + 1)
            return carry
        lax.fori_loop(0, NBLKA // 2, body2, 0)
        wait_scatter(sA, 0, semSA)
        wait_scatter(sB, 1, semSB)
        plsc.subcore_barrier()

        pltpu.sync_copy(hsh.at[pl.ds(base_r, SR)],
                        out_hbm.at[c, pl.ds(base_r, SR)])

        @pl.when(s == NS - 1)
        def _write_tail():
            pltpu.sync_copy(hsh.at[pl.ds(NS * SR, TAIL)],
                            out_hbm.at[c, pl.ds(NS * SR, TAIL)])

    return agg(support, row, col, vals)


def _sc_decode(z, src, dst):
    mesh = plsc.VectorSubcoreMesh(**_MESH)

    @functools.partial(
        pl.kernel,
        mesh=mesh,
        compiler_params=pltpu.CompilerParams(needs_layout_passes=False),
        out_type=jax.ShapeDtypeStruct((E,), jnp.float32),
        scratch_types=[
            pltpu.VMEM((EPW,), jnp.int32),
            pltpu.VMEM((EPW,), jnp.int32),
            pltpu.VMEM((BK, D), jnp.float32),
            pltpu.VMEM((BK, D), jnp.float32),
            pltpu.VMEM((BK, D), jnp.float32),
            pltpu.VMEM((BK, D), jnp.float32),
            pltpu.VMEM((16, 16), jnp.float32),
            pltpu.VMEM((BK,), jnp.float32),
            pltpu.SemaphoreType.DMA,
            pltpu.SemaphoreType.DMA,
            pltpu.SemaphoreType.DMA,
        ],
    )
    def dec(z_hbm, src_hbm, dst_hbm, out_hbm,
            sv, dv, zaA, zbA, zaB, zbB, pbuf, res, semA, semB, semI):
        c = lax.axis_index("c")
        s = lax.axis_index("s")
        wid = c * NS + s
        e0 = wid * EPW
        lanes = lax.iota(jnp.int32, 16)

        pltpu.async_copy(src_hbm.at[pl.ds(e0, EPW)], sv, semI)
        pltpu.async_copy(dst_hbm.at[pl.ds(e0, EPW)], dv, semI)
        pltpu.make_async_copy(src_hbm.at[pl.ds(e0, EPW)], sv, semI).wait()
        pltpu.make_async_copy(dst_hbm.at[pl.ds(e0, EPW)], dv, semI).wait()

        def issue(za, zb, sem, blk_i):
            pltpu.async_copy(z_hbm.at[sv.at[pl.ds(blk_i * BK, BK)]], za, sem)
            pltpu.async_copy(z_hbm.at[dv.at[pl.ds(blk_i * BK, BK)]], zb, sem)

        def drain(za, zb, sem, blk_i):
            pltpu.make_async_copy(
                z_hbm.at[sv.at[pl.ds(blk_i * BK, BK)]], za, sem).wait()
            pltpu.make_async_copy(
                z_hbm.at[dv.at[pl.ds(blk_i * BK, BK)]], zb, sem).wait()

        def compute(za, zb, blk_i):
            def grp(g5, carry):
                for t in range(16):
                    j = g5 * 16 + t
                    acc = za[j, pl.ds(0, 16)] * zb[j, pl.ds(0, 16)]
                    for k in range(1, D // 16):
                        sl = pl.ds(k * 16, 16)
                        acc = acc + za[j, sl] * zb[j, sl]
                    pbuf[t, pl.ds(0, 16)] = acc
                tot = plsc.load_gather(pbuf, [lanes, jnp.zeros((16,), jnp.int32)])
                for d2 in range(1, 16):
                    tot = tot + plsc.load_gather(
                        pbuf, [lanes, jnp.full((16,), d2, jnp.int32)])
                res[pl.ds(g5 * 16, 16)] = tot
                return carry
            lax.fori_loop(0, BK // 16, grp, 0)
            pltpu.sync_copy(res, out_hbm.at[pl.ds(e0 + blk_i * BK, BK)])

        issue(zaA, zbA, semA, 0)
        issue(zaB, zbB, semB, 1)

        def body2(i, carry):
            bA = 2 * i
            drain(zaA, zbA, semA, bA)
            compute(zaA, zbA, bA)
            issue(zaA, zbA, semA, bA + 2)
            drain(zaB, zbB, semB, bA + 1)
            compute(zaB, zbB, bA + 1)

            @pl.when(i < (NBLK - 3) // 2)
            def _next_b():
                issue(zaB, zbB, semB, bA + 3)
            return carry
        lax.fori_loop(0, (NBLK - 1) // 2, body2, 0)
        drain(zaA, zbA, semA, NBLK - 1)
        compute(zaA, zbA, NBLK - 1)

    return dec(z, src, dst)


def kernel(x, adj_sparse_indices, adj_sparse_values, edge_index, W1, b1, W2, b2):
    row = adj_sparse_indices[0]
    col = adj_sparse_indices[1]
    src = edge_index[0]
    dst = edge_index[1]

    support = _linear(x, W1, b1)
    hp = _sc_aggregate(support, row, col, adj_sparse_values)
    support2 = _combine_relu_linear(hp, W2, b2)
    zp = _sc_aggregate(support2, row, col, adj_sparse_values)
    z = _combine(zp)
    return _sc_decode(z, src, dst)
